# Initial kernel scaffold; baseline (speedup 1.0000x reference)
import jax
import jax.numpy as jnp
from jax.experimental import pallas as pl


def kernel(node_representation, graph_index, W, b):
    raise NotImplementedError("write your pallas kernel here")


# speedup vs baseline: 4.9728x; 4.9728x over previous
import functools

import jax
import jax.numpy as jnp
from jax import lax
from jax.experimental import pallas as pl
from jax.experimental.pallas import tpu as pltpu
from jax.experimental.pallas import tpu_sc as plsc

N_NODES = 100000
D_FEAT = 128
NUM_GRAPHS = 512
D_OUT = 128

_NC = 2
_NS = 16
_NW = _NC * _NS
_SUB = 80
_NSUB = 2
_GC = _SUB * _NSUB
_NCHUNKS = N_NODES // _GC
_Q, _R = divmod(_NCHUNKS, _NW)
_QMAX = _Q + 1
_NCHUNKS_PAD = _QMAX * _NW
_LANES = 16
_GROWS = NUM_GRAPHS // _NS


def _make_seg_pool(cw):
  mesh = plsc.VectorSubcoreMesh(core_axis_name="c", subcore_axis_name="s")

  @functools.partial(
      pl.kernel,
      mesh=mesh,
      out_type=(
          jax.ShapeDtypeStruct((_NC, NUM_GRAPHS, D_FEAT), jnp.float32),
          jax.ShapeDtypeStruct((_NC, NUM_GRAPHS, cw), jnp.float32),
      ),
      scratch_types=(
          pltpu.VMEM((_GC, D_FEAT), jnp.float32),
          pltpu.VMEM((_QMAX, _NSUB, _SUB), jnp.int32),
          pltpu.VMEM((_GROWS, D_FEAT), jnp.float32),
          pltpu.VMEM((_SUB, cw), jnp.float32),
          pltpu.VMEM((_GROWS, cw), jnp.float32),
          pltpu.VMEM_SHARED((NUM_GRAPHS, D_FEAT), jnp.float32),
          pltpu.VMEM_SHARED((NUM_GRAPHS, cw), jnp.float32),
      ),
  )
  def k(nodes_hbm, idx_hbm, zrow_hbm, zcnt_hbm, ones_hbm, sums_hbm, cnts_hbm,
        rows_v, idx_v, zrow_v, ones_v, zcnt_v, sums_sh, cnts_sh):
    cid = lax.axis_index("c")
    sid = lax.axis_index("s")
    wid = sid * _NC + cid

    pltpu.sync_copy(zrow_hbm, zrow_v)
    pltpu.sync_copy(zcnt_hbm, zcnt_v)
    pltpu.sync_copy(ones_hbm, ones_v)

    pltpu.sync_copy(zrow_v, sums_sh.at[pl.ds(sid * _GROWS, _GROWS)])
    pltpu.sync_copy(zcnt_v, cnts_sh.at[pl.ds(sid * _GROWS, _GROWS)])
    plsc.subcore_barrier()

    start = wid * _Q + jnp.minimum(wid, _R)
    count = _Q + (wid < _R).astype(jnp.int32)

    pltpu.sync_copy(idx_hbm.at[pl.ds(start, _QMAX)], idx_v)

    def body(kk, _):
      base = (start + kk) * _GC
      pltpu.sync_copy(nodes_hbm.at[pl.ds(base, _GC)], rows_v)
      for j in range(_NSUB):
        pltpu.sync_copy(rows_v.at[pl.ds(j * _SUB, _SUB)],
                        sums_sh.at[idx_v.at[kk, j]], add=True)
      return 0
    lax.fori_loop(0, count, body, 0)

    def cbody(kk, _):
      for j in range(_NSUB):
        pltpu.sync_copy(ones_v, cnts_sh.at[idx_v.at[kk, j]], add=True)
      return 0
    lax.fori_loop(0, count, cbody, 0)

    plsc.subcore_barrier()
    pltpu.sync_copy(sums_sh.at[pl.ds(sid * _GROWS, _GROWS)],
                    sums_hbm.at[cid, pl.ds(sid * _GROWS, _GROWS)])
    pltpu.sync_copy(cnts_sh.at[pl.ds(sid * _GROWS, _GROWS)],
                    cnts_hbm.at[cid, pl.ds(sid * _GROWS, _GROWS)])

  def run(nodes, idx3):
    zrow = jnp.zeros((_GROWS, D_FEAT), jnp.float32)
    zcnt = jnp.zeros((_GROWS, cw), jnp.float32)
    ones = jnp.ones((_SUB, cw), jnp.float32)
    return k(nodes, idx3, zrow, zcnt, ones)

  return run


_CW = 128
_seg_pool = _make_seg_pool(_CW)


def _head_body(ps_ref, pc_ref, w_ref, b_ref, o_ref):
  s = ps_ref[0] + ps_ref[1]
  c = pc_ref[0, :, 0:1] + pc_ref[1, :, 0:1]
  pooled = s / jnp.maximum(c, 1.0)
  o_ref[...] = lax.dot_general(
      pooled, w_ref[...], (((1,), (1,)), ((), ())),
      preferred_element_type=jnp.float32) + b_ref[...]


def kernel(node_representation, graph_index, W, b):
  idx3 = graph_index.astype(jnp.int32).reshape(_NCHUNKS, _NSUB, _SUB)
  idx3 = jnp.pad(idx3, ((0, _NCHUNKS_PAD - _NCHUNKS), (0, 0), (0, 0)))
  sums, cnts = _seg_pool(node_representation, idx3)
  out = pl.pallas_call(
      _head_body,
      out_shape=jax.ShapeDtypeStruct((NUM_GRAPHS, D_OUT), jnp.float32),
  )(sums, cnts, W, b.reshape(1, D_OUT))
  return out


# speedup vs baseline: 5.9698x; 1.2005x over previous
import functools

import jax
import jax.numpy as jnp
from jax import lax
from jax.experimental import pallas as pl
from jax.experimental.pallas import tpu as pltpu
from jax.experimental.pallas import tpu_sc as plsc

N_NODES = 100000
D_FEAT = 128
NUM_GRAPHS = 512
D_OUT = 128

_NC = 2
_NS = 16
_NW = _NC * _NS
_SUB = 80
_NSUB = 2
_GC = _SUB * _NSUB
_NCHUNKS = N_NODES // _GC
_Q, _R = divmod(_NCHUNKS, _NW)
_QMAX = _Q + 1
_NCHUNKS_PAD = _QMAX * _NW
_LANES = 16
_GROWS = NUM_GRAPHS // _NS


def _make_seg_pool(cw):
  mesh = plsc.VectorSubcoreMesh(core_axis_name="c", subcore_axis_name="s")

  @functools.partial(
      pl.kernel,
      mesh=mesh,
      out_type=(
          jax.ShapeDtypeStruct((_NC, NUM_GRAPHS, D_FEAT), jnp.float32),
          jax.ShapeDtypeStruct((_NC, NUM_GRAPHS, cw), jnp.float32),
      ),
      scratch_types=(
          pltpu.VMEM((2, _GC, D_FEAT), jnp.float32),
          pltpu.VMEM((_QMAX, _NSUB, _SUB), jnp.int32),
          pltpu.VMEM((_GROWS, D_FEAT), jnp.float32),
          pltpu.VMEM((_SUB, cw), jnp.float32),
          pltpu.VMEM((_GROWS, cw), jnp.float32),
          pltpu.VMEM_SHARED((NUM_GRAPHS, D_FEAT), jnp.float32),
          pltpu.VMEM_SHARED((NUM_GRAPHS, cw), jnp.float32),
          pltpu.SemaphoreType.DMA,
          pltpu.SemaphoreType.DMA,
      ),
  )
  def k(nodes_hbm, idx_hbm, zrow_hbm, zcnt_hbm, ones_hbm, sums_hbm, cnts_hbm,
        rows_v, idx_v, zrow_v, ones_v, zcnt_v, sums_sh, cnts_sh, sem0, sem1):
    cid = lax.axis_index("c")
    sid = lax.axis_index("s")
    wid = sid * _NC + cid

    pltpu.sync_copy(zrow_hbm, zrow_v)
    pltpu.sync_copy(zcnt_hbm, zcnt_v)
    pltpu.sync_copy(ones_hbm, ones_v)

    pltpu.sync_copy(zrow_v, sums_sh.at[pl.ds(sid * _GROWS, _GROWS)])
    pltpu.sync_copy(zcnt_v, cnts_sh.at[pl.ds(sid * _GROWS, _GROWS)])
    plsc.subcore_barrier()

    start = wid * _Q + jnp.minimum(wid, _R)
    count = _Q + (wid < _R).astype(jnp.int32)

    pltpu.sync_copy(idx_hbm.at[pl.ds(start, _QMAX)], idx_v)

    sems = (sem0, sem1)

    def gather(kk, b):
      return pltpu.make_async_copy(
          nodes_hbm.at[pl.ds((start + kk) * _GC, _GC)], rows_v.at[b], sems[b])

    def issue(kk, b):
      pl.when(kk < count)(lambda: gather(kk, b).start())

    issue(0, 0)
    issue(1, 1)

    def body(g, _):
      for b in range(2):
        kk = 2 * g + b

        def work(kk=kk, b=b):
          gather(kk, b).wait()
          for j in range(_NSUB):
            pltpu.sync_copy(rows_v.at[b, pl.ds(j * _SUB, _SUB)],
                            sums_sh.at[idx_v.at[kk, j]], add=True)
          issue(kk + 2, b)
        pl.when(kk < count)(work)
      return 0
    lax.fori_loop(0, _QMAX // 2, body, 0)

    def cbody(kk, _):
      for j in range(_NSUB):
        pltpu.sync_copy(ones_v, cnts_sh.at[idx_v.at[kk, j]], add=True)
      return 0
    lax.fori_loop(0, count, cbody, 0)

    plsc.subcore_barrier()
    pltpu.sync_copy(sums_sh.at[pl.ds(sid * _GROWS, _GROWS)],
                    sums_hbm.at[cid, pl.ds(sid * _GROWS, _GROWS)])
    pltpu.sync_copy(cnts_sh.at[pl.ds(sid * _GROWS, _GROWS)],
                    cnts_hbm.at[cid, pl.ds(sid * _GROWS, _GROWS)])

  def run(nodes, idx3):
    zrow = jnp.zeros((_GROWS, D_FEAT), jnp.float32)
    zcnt = jnp.zeros((_GROWS, cw), jnp.float32)
    ones = jnp.ones((_SUB, cw), jnp.float32)
    return k(nodes, idx3, zrow, zcnt, ones)

  return run


_CW = 128
_seg_pool = _make_seg_pool(_CW)


def _head_body(ps_ref, pc_ref, w_ref, b_ref, o_ref):
  s = ps_ref[0] + ps_ref[1]
  c = pc_ref[0, :, 0:1] + pc_ref[1, :, 0:1]
  pooled = s / jnp.maximum(c, 1.0)
  o_ref[...] = lax.dot_general(
      pooled, w_ref[...], (((1,), (1,)), ((), ())),
      preferred_element_type=jnp.float32) + b_ref[...]


def kernel(node_representation, graph_index, W, b):
  idx3 = graph_index.astype(jnp.int32).reshape(_NCHUNKS, _NSUB, _SUB)
  idx3 = jnp.pad(idx3, ((0, _NCHUNKS_PAD - _NCHUNKS), (0, 0), (0, 0)))
  sums, cnts = _seg_pool(node_representation, idx3)
  out = pl.pallas_call(
      _head_body,
      out_shape=jax.ShapeDtypeStruct((NUM_GRAPHS, D_OUT), jnp.float32),
  )(sums, cnts, W, b.reshape(1, D_OUT))
  return out
